# baseline (speedup 1.0000x reference)
import functools

import jax
import jax.numpy as jnp
from jax import lax
from jax.experimental import pallas as pl
from jax.experimental.pallas import tpu as pltpu
from jax.experimental.pallas import tpu_sc as plsc

_B = 4096
_E = 128
_S = _B // _E
_NEG = 20
_NC, _NS = 2, 16
_NW = _NC * _NS
_PER = _B // _NW
_TCG = 32


def _sc_gather_body(ww_hbm, wc_hbm, x_hbm, y_hbm, neg_hbm,
                    outx_hbm, outxp_hbm, outyg_hbm, outn_hbm,
                    idxx_v, idxy_v, idxn_v, didx_v, gx_v, gy_v, gn_v,
                    sem_x, sem_y, sem_n, sem_o):
    wid = lax.axis_index("s") * _NC + lax.axis_index("c")
    base = wid * _PER
    ld_x = pltpu.async_copy(x_hbm.at[pl.ds(base, _PER)], idxx_v, sem_x)
    ld_y = pltpu.async_copy(y_hbm.at[pl.ds(base, _PER)], idxy_v, sem_y)
    ld_x.wait()
    cp_x = pltpu.async_copy(ww_hbm.at[idxx_v], gx_v, sem_x)
    ld_y.wait()
    cp_y = pltpu.async_copy(wc_hbm.at[idxy_v], gy_v, sem_y)

    lane = lax.broadcasted_iota(jnp.int32, (16,), 0) * 128
    for t in range(_PER // 16):
        didx_v[pl.ds(t * 16, 16)] = lane + (2048 * (t % 2) + 4 * wid + t // 2)

    @pl.when(wid == 0)
    def _neg_path():
        pltpu.sync_copy(neg_hbm, idxn_v)
        pltpu.async_copy(wc_hbm.at[idxn_v], gn_v, sem_n).wait()
        pltpu.sync_copy(gn_v, outn_hbm)

    cp_x.wait()
    wb_xn = pltpu.async_copy(gx_v, outx_hbm.at[pl.ds(base, _PER)], sem_o)
    wb_xp = pltpu.async_copy(gx_v, outxp_hbm.at[didx_v], sem_o)
    cp_y.wait()
    wb_yg = pltpu.async_copy(gy_v, outyg_hbm.at[didx_v], sem_o)
    wb_xn.wait()
    wb_xp.wait()
    wb_yg.wait()


@functools.cache
def _sc_gather():
    mesh = plsc.VectorSubcoreMesh(core_axis_name="c", subcore_axis_name="s",
                                  num_cores=_NC, num_subcores=_NS)
    return pl.kernel(
        _sc_gather_body,
        out_type=(
            jax.ShapeDtypeStruct((_B, _E), jnp.float32),
            jax.ShapeDtypeStruct((_B, _E), jnp.float32),
            jax.ShapeDtypeStruct((_B, _E), jnp.float32),
            jax.ShapeDtypeStruct((_NEG, _E), jnp.float32),
        ),
        mesh=mesh,
        scratch_types=[
            pltpu.VMEM((_PER,), jnp.int32),
            pltpu.VMEM((_PER,), jnp.int32),
            pltpu.VMEM((_NEG,), jnp.int32),
            pltpu.VMEM((_PER,), jnp.int32),
            pltpu.VMEM((_PER, _E), jnp.float32),
            pltpu.VMEM((_PER, _E), jnp.float32),
            pltpu.VMEM((_NEG, _E), jnp.float32),
            pltpu.SemaphoreType.DMA,
            pltpu.SemaphoreType.DMA,
            pltpu.SemaphoreType.DMA,
            pltpu.SemaphoreType.DMA,
        ],
    )


def _nls(z):
    return jnp.maximum(-z, 0.0) + jnp.log(1.0 + jnp.exp(-jnp.abs(z)))


def _tc_body(xn_ref, xp_ref, yg_ref, ng_ref, out_ref):
    g = pl.program_id(0)
    ng = ng_ref[...]
    eye = (lax.broadcasted_iota(jnp.int32, (_E, _E), 0)
           == lax.broadcasted_iota(jnp.int32, (_E, _E), 1))
    acc = jnp.float32(0.0)
    for j in range(_TCG):
        sl = pl.ds(j * _E, _E)
        xn = xn_ref[sl, :]
        xp = xp_ref[sl, :]
        yg = yg_ref[sl, :]
        m = jnp.dot(xn.astype(jnp.bfloat16), yg.astype(jnp.bfloat16),
                    preferred_element_type=jnp.float32)
        diag = jnp.sum(jnp.where(eye, m, 0.0), axis=0, keepdims=True)
        nb = jnp.dot(ng.astype(jnp.bfloat16), xp.astype(jnp.bfloat16),
                     preferred_element_type=jnp.float32)
        acc += jnp.sum(_nls(diag)) * (1.0 / _B) + jnp.sum(_nls(-nb))

    @pl.when(g == 0)
    def _init():
        out_ref[0, 0] = 0.0

    out_ref[0, 0] += acc


def _tc_reduce(outx, outxp, outyg, outn):
    blk = _TCG * _E
    return pl.pallas_call(
        _tc_body,
        grid=(_S // _TCG,),
        in_specs=[
            pl.BlockSpec((blk, _E), lambda g: (g, 0)),
            pl.BlockSpec((blk, _E), lambda g: (g, 0)),
            pl.BlockSpec((blk, _E), lambda g: (g, 0)),
            pl.BlockSpec((_NEG, _E), lambda g: (0, 0)),
        ],
        out_specs=pl.BlockSpec((1, 1), lambda g: (0, 0),
                               memory_space=pltpu.SMEM),
        out_shape=jax.ShapeDtypeStruct((1, 1), jnp.float32),
    )(outx, outxp, outyg, outn)


def kernel(x, y, neg_samples, W_word, W_ctx):
    x = x.astype(jnp.int32)
    y = y.astype(jnp.int32)
    neg = neg_samples.astype(jnp.int32)
    outx, outxp, outyg, outn = _sc_gather()(W_word, W_ctx, x, y, neg)
    res = _tc_reduce(outx, outxp, outyg, outn)
    return res[0, 0]


# speedup vs baseline: 1.0233x; 1.0233x over previous
import functools

import jax
import jax.numpy as jnp
from jax import lax
from jax.experimental import pallas as pl
from jax.experimental.pallas import tpu as pltpu
from jax.experimental.pallas import tpu_sc as plsc

_B = 4096
_E = 128
_S = _B // _E
_NEG = 20
_NC, _NS = 2, 16
_NW = _NC * _NS
_PER = _B // _NW
_TCG = 32


def _sc_gather_body(ww_hbm, wc_hbm, x_hbm, y_hbm, neg_hbm,
                    outxp_hbm, outyg_hbm, outn_hbm,
                    idxx_v, idxy_v, idxn_v, didx_v, gx_v, gy_v, gn_v,
                    sem_x, sem_y, sem_n, sem_o):
    wid = lax.axis_index("s") * _NC + lax.axis_index("c")
    base = wid * _PER
    ld_x = pltpu.async_copy(x_hbm.at[pl.ds(base, _PER)], idxx_v, sem_x)
    ld_y = pltpu.async_copy(y_hbm.at[pl.ds(base, _PER)], idxy_v, sem_y)
    ld_x.wait()
    cp_x = pltpu.async_copy(ww_hbm.at[idxx_v], gx_v, sem_x)
    ld_y.wait()
    cp_y = pltpu.async_copy(wc_hbm.at[idxy_v], gy_v, sem_y)

    lane = lax.broadcasted_iota(jnp.int32, (16,), 0) * 128
    for t in range(_PER // 16):
        didx_v[pl.ds(t * 16, 16)] = lane + (2048 * (t % 2) + 4 * wid + t // 2)

    @pl.when(wid == 0)
    def _neg_path():
        pltpu.sync_copy(neg_hbm, idxn_v)
        pltpu.async_copy(wc_hbm.at[idxn_v], gn_v, sem_n).wait()
        pltpu.sync_copy(gn_v, outn_hbm)

    cp_x.wait()
    wb_xp = pltpu.async_copy(gx_v, outxp_hbm.at[didx_v], sem_o)
    cp_y.wait()
    wb_yg = pltpu.async_copy(gy_v, outyg_hbm.at[didx_v], sem_o)
    wb_xp.wait()
    wb_yg.wait()


@functools.cache
def _sc_gather():
    mesh = plsc.VectorSubcoreMesh(core_axis_name="c", subcore_axis_name="s",
                                  num_cores=_NC, num_subcores=_NS)
    return pl.kernel(
        _sc_gather_body,
        out_type=(
            jax.ShapeDtypeStruct((_B, _E), jnp.float32),
            jax.ShapeDtypeStruct((_B, _E), jnp.float32),
            jax.ShapeDtypeStruct((_NEG, _E), jnp.float32),
        ),
        mesh=mesh,
        scratch_types=[
            pltpu.VMEM((_PER,), jnp.int32),
            pltpu.VMEM((_PER,), jnp.int32),
            pltpu.VMEM((_NEG,), jnp.int32),
            pltpu.VMEM((_PER,), jnp.int32),
            pltpu.VMEM((_PER, _E), jnp.float32),
            pltpu.VMEM((_PER, _E), jnp.float32),
            pltpu.VMEM((_NEG, _E), jnp.float32),
            pltpu.SemaphoreType.DMA,
            pltpu.SemaphoreType.DMA,
            pltpu.SemaphoreType.DMA,
            pltpu.SemaphoreType.DMA,
        ],
    )


def _nls(z):
    return jnp.maximum(-z, 0.0) + jnp.log(1.0 + jnp.exp(-jnp.abs(z)))


def _tc_body(xp_ref, yg_ref, ng_ref, out_ref):
    ng = ng_ref[...].astype(jnp.bfloat16)
    xpv = xp_ref[...]
    xp4 = xpv.reshape(_S, _S // 2, 8, _E)
    pi = lax.broadcasted_iota(jnp.int32, (2 * _E, _E), 0)
    ci = lax.broadcasted_iota(jnp.int32, (2 * _E, _E), 1)
    v = pi % 8
    s = pi // 8
    mask0 = (v < 4) & (ci == 32 * v + s)
    mask1 = (v >= 4) & (ci == 32 * (v - 4) + s)
    acc = jnp.float32(0.0)
    for gg in range(_S // 2):
        fb = xp4[:, gg, :, :].reshape(2 * _E, _E).astype(jnp.bfloat16)
        yg0 = yg_ref[pl.ds((2 * gg) * _E, _E), :].astype(jnp.bfloat16)
        yg1 = yg_ref[pl.ds((2 * gg + 1) * _E, _E), :].astype(jnp.bfloat16)
        m0 = jnp.dot(fb, yg0, preferred_element_type=jnp.float32)
        m1 = jnp.dot(fb, yg1, preferred_element_type=jnp.float32)
        d0 = jnp.sum(jnp.where(mask0, m0, 0.0), axis=0, keepdims=True)
        d1 = jnp.sum(jnp.where(mask1, m1, 0.0), axis=0, keepdims=True)
        nb0 = jnp.dot(
            ng, xpv[(2 * gg) * _E:(2 * gg + 1) * _E, :].astype(jnp.bfloat16),
            preferred_element_type=jnp.float32)
        nb1 = jnp.dot(
            ng, xpv[(2 * gg + 1) * _E:(2 * gg + 2) * _E, :].astype(
                jnp.bfloat16),
            preferred_element_type=jnp.float32)
        acc += ((jnp.sum(_nls(d0)) + jnp.sum(_nls(d1))) * (1.0 / _B)
                + jnp.sum(_nls(-nb0)) + jnp.sum(_nls(-nb1)))
    out_ref[0, 0] = acc


def _tc_reduce(outxp, outyg, outn):
    return pl.pallas_call(
        _tc_body,
        in_specs=[
            pl.BlockSpec((_B, _E), lambda: (0, 0)),
            pl.BlockSpec((_B, _E), lambda: (0, 0)),
            pl.BlockSpec((_NEG, _E), lambda: (0, 0)),
        ],
        out_specs=pl.BlockSpec((1, 1), lambda: (0, 0),
                               memory_space=pltpu.SMEM),
        out_shape=jax.ShapeDtypeStruct((1, 1), jnp.float32),
    )(outxp, outyg, outn)


def kernel(x, y, neg_samples, W_word, W_ctx):
    x = x.astype(jnp.int32)
    y = y.astype(jnp.int32)
    neg = neg_samples.astype(jnp.int32)
    outxp, outyg, outn = _sc_gather()(W_word, W_ctx, x, y, neg)
    res = _tc_reduce(outxp, outyg, outn)
    return res[0, 0]
